# baseline (speedup 1.0000x reference)
import functools

import jax
import jax.numpy as jnp
from jax import lax
from jax.experimental import pallas as pl
from jax.experimental.pallas import tpu as pltpu
from jax.experimental.pallas import tpu_sc as plsc

_NUM_FIELDS = 26
_VOCAB = 100000
_D = 100
_DP = 128
_BATCH = 16384
_B_TOTAL = _BATCH * _NUM_FIELDS
_NC = 2
_NS = 16
_NW = _NC * _NS
_ROWS_PER_W = _B_TOTAL // _NW
_CHUNK = 128
_N_CHUNKS = _ROWS_PER_W // _CHUNK

_mesh = plsc.VectorSubcoreMesh(core_axis_name="c", subcore_axis_name="s")


@functools.partial(
    pl.kernel,
    out_type=jax.ShapeDtypeStruct((_B_TOTAL, _DP), jnp.float32),
    mesh=_mesh,
    scratch_types=[
        pltpu.VMEM((_N_CHUNKS, _CHUNK), jnp.int32),
        pltpu.VMEM((2, _CHUNK, _DP), jnp.float32),
        pltpu.SemaphoreType.DMA,
        pltpu.SemaphoreType.DMA,
    ],
)
def _sc_gather(table_hbm, idx_hbm, out_hbm, idx_v, rows_v, gsem, ssem):
    wid = lax.axis_index("s") * _NC + lax.axis_index("c")
    base = wid * _ROWS_PER_W
    pltpu.sync_copy(idx_hbm.at[wid], idx_v)

    @pl.loop(0, _N_CHUNKS)
    def _chunk(j):
        pltpu.async_copy(table_hbm.at[idx_v.at[j]], rows_v.at[0], gsem).wait()
        pltpu.sync_copy(rows_v.at[0], out_hbm.at[pl.ds(base + j * _CHUNK, _CHUNK)])


_PAD_BLK = 1000


def _pad_body(src_ref, dst_ref):
    dst_ref[:, : _D] = src_ref[...]


_tc_pad = pl.pallas_call(
    _pad_body,
    out_shape=jax.ShapeDtypeStruct((_NUM_FIELDS * _VOCAB, _DP), jnp.float32),
    grid=(_NUM_FIELDS * _VOCAB // _PAD_BLK,),
    in_specs=[pl.BlockSpec((_PAD_BLK, _D), lambda i: (i, 0))],
    out_specs=pl.BlockSpec((_PAD_BLK, _DP), lambda i: (i, 0)),
)


def kernel(x_cat, tables):
    flat_table = tables.reshape(_NUM_FIELDS * _VOCAB, _D)
    padded = _tc_pad(flat_table)
    offs = jnp.arange(_NUM_FIELDS, dtype=jnp.int32) * _VOCAB
    flat_idx = (x_cat.astype(jnp.int32) + offs[None, :]).reshape(
        _NW, _N_CHUNKS, _CHUNK
    )
    out = _sc_gather(padded, flat_idx)
    return out[:, :_D].reshape(_BATCH, _NUM_FIELDS * _D)


# speedup vs baseline: 4.7399x; 4.7399x over previous
import functools

import jax
import jax.numpy as jnp
from jax import lax
from jax.experimental import pallas as pl
from jax.experimental.pallas import tpu as pltpu
from jax.experimental.pallas import tpu_sc as plsc

_NUM_FIELDS = 26
_FG = 13
_VOCAB = 100000
_D = 100
_DP = 128
_BATCH = 16384
_BT_G = _BATCH * _FG
_NC = 2
_NS = 16
_NW = _NC * _NS
_ROWS_PER_W = _BT_G // _NW
_CHUNK = 128
_N_CHUNKS = _ROWS_PER_W // _CHUNK

_mesh = plsc.VectorSubcoreMesh(core_axis_name="c", subcore_axis_name="s")


@functools.partial(
    pl.kernel,
    out_type=jax.ShapeDtypeStruct((_BT_G, _DP), jnp.float32),
    mesh=_mesh,
    scratch_types=[
        pltpu.VMEM((_N_CHUNKS, _CHUNK), jnp.int32),
        pltpu.VMEM((2, _CHUNK, _DP), jnp.float32),
        pltpu.SemaphoreType.DMA,
        pltpu.SemaphoreType.DMA,
    ],
)
def _sc_gather(table_hbm, idx_hbm, out_hbm, idx_v, rows_v, g0, g1):
    wid = lax.axis_index("s") * _NC + lax.axis_index("c")
    base = wid * _ROWS_PER_W
    pltpu.sync_copy(idx_hbm.at[wid], idx_v)

    gsems = (g0, g1)

    def _gather(j, t):
        pltpu.async_copy(table_hbm.at[idx_v.at[j]], rows_v.at[t], gsems[t])

    def _wait(t):
        pltpu.make_async_copy(
            table_hbm.at[pl.ds(0, _CHUNK)], rows_v.at[t], gsems[t]
        ).wait()

    def _store(j, t):
        pltpu.sync_copy(rows_v.at[t], out_hbm.at[pl.ds(base + j * _CHUNK, _CHUNK)])

    _gather(0, 0)

    @pl.loop(0, _N_CHUNKS, step=2)
    def _chunk(j):
        _gather(j + 1, 1)
        _wait(0)
        _store(j, 0)

        @pl.when(j + 2 < _N_CHUNKS)
        def _():
            _gather(j + 2, 0)

        _wait(1)
        _store(j + 1, 1)


_VBLK = 16384
_NVB = (_VOCAB + _VBLK - 1) // _VBLK


def _tr_body(src_ref, dst_ref):
    dst_ref[0, :, : _D] = jnp.swapaxes(src_ref[0], 0, 1)


def _make_tc_transpose(f0):
    return pl.pallas_call(
        _tr_body,
        out_shape=jax.ShapeDtypeStruct((_FG, _VOCAB, _DP), jnp.float32),
        grid=(_FG, _NVB),
        in_specs=[pl.BlockSpec((1, _D, _VBLK), lambda f, j: (f + f0, 0, j))],
        out_specs=pl.BlockSpec((1, _VBLK, _DP), lambda f, j: (f, j, 0)),
    )


_tc_transpose_lo = _make_tc_transpose(0)
_tc_transpose_hi = _make_tc_transpose(_FG)


def _group_idx(x_cat, f0):
    offs = jnp.arange(_FG, dtype=jnp.int32) * _VOCAB
    cols = x_cat[:, f0 : f0 + _FG].astype(jnp.int32)
    return (cols + offs[None, :]).reshape(_NW, _N_CHUNKS, _CHUNK)


def kernel(x_cat, tables):
    tables_t = jnp.transpose(tables, (0, 2, 1))
    pad_lo = _tc_transpose_lo(tables_t).reshape(_FG * _VOCAB, _DP)
    out_lo = _sc_gather(pad_lo, _group_idx(x_cat, 0))
    pad_hi = _tc_transpose_hi(tables_t).reshape(_FG * _VOCAB, _DP)
    out_hi = _sc_gather(pad_hi, _group_idx(x_cat, _FG))
    lo = out_lo[:, :_D].reshape(_BATCH, _FG * _D)
    hi = out_hi[:, :_D].reshape(_BATCH, _FG * _D)
    return jnp.concatenate([lo, hi], axis=1)


# speedup vs baseline: 5.1882x; 1.0946x over previous
import functools

import jax
import jax.numpy as jnp
from jax import lax
from jax.experimental import pallas as pl
from jax.experimental.pallas import tpu as pltpu
from jax.experimental.pallas import tpu_sc as plsc

_NUM_FIELDS = 26
_VOCAB = 100000
_D = 100
_DP = 128
_BATCH = 16384
_B_TOTAL = _BATCH * _NUM_FIELDS
_NC = 2
_NS = 16
_NW = _NC * _NS
_ROWS_PER_W = _B_TOTAL // _NW
_CHUNK = 128
_N_CHUNKS = _ROWS_PER_W // _CHUNK

_mesh = plsc.VectorSubcoreMesh(core_axis_name="c", subcore_axis_name="s")


@functools.partial(
    pl.kernel,
    out_type=jax.ShapeDtypeStruct((_B_TOTAL, _DP), jnp.float32),
    mesh=_mesh,
    scratch_types=[
        pltpu.VMEM((_N_CHUNKS, _CHUNK), jnp.int32),
        pltpu.VMEM((2, _CHUNK, _DP), jnp.float32),
        pltpu.SemaphoreType.DMA,
        pltpu.SemaphoreType.DMA,
    ],
)
def _sc_gather(table_hbm, idx_hbm, out_hbm, idx_v, rows_v, g0, g1):
    wid = lax.axis_index("s") * _NC + lax.axis_index("c")
    base = wid * _ROWS_PER_W
    pltpu.sync_copy(idx_hbm.at[wid], idx_v)

    gsems = (g0, g1)

    def _gather(j, t):
        pltpu.async_copy(table_hbm.at[idx_v.at[j]], rows_v.at[t], gsems[t])

    def _wait(t):
        pltpu.make_async_copy(
            table_hbm.at[pl.ds(0, _CHUNK)], rows_v.at[t], gsems[t]
        ).wait()

    def _store(j, t):
        pltpu.sync_copy(rows_v.at[t], out_hbm.at[pl.ds(base + j * _CHUNK, _CHUNK)])

    _gather(0, 0)

    @pl.loop(0, _N_CHUNKS, step=2)
    def _chunk(j):
        _gather(j + 1, 1)
        _wait(0)
        _store(j, 0)

        @pl.when(j + 2 < _N_CHUNKS)
        def _():
            _gather(j + 2, 0)

        _wait(1)
        _store(j + 1, 1)


_VBLK = 32768
_NVB = (_VOCAB + _VBLK - 1) // _VBLK


def _tr_body(src_ref, dst_ref):
    dst_ref[0, :, : _D] = jnp.swapaxes(src_ref[0], 0, 1)


_tc_transpose = pl.pallas_call(
    _tr_body,
    out_shape=jax.ShapeDtypeStruct((_NUM_FIELDS, _VOCAB, _DP), jnp.float32),
    grid=(_NUM_FIELDS, _NVB),
    in_specs=[pl.BlockSpec((1, _D, _VBLK), lambda f, j: (f, 0, j))],
    out_specs=pl.BlockSpec((1, _VBLK, _DP), lambda f, j: (f, j, 0)),
)


def kernel(x_cat, tables):
    tables_t = jnp.transpose(tables, (0, 2, 1))
    padded = _tc_transpose(tables_t).reshape(_NUM_FIELDS * _VOCAB, _DP)
    offs = jnp.arange(_NUM_FIELDS, dtype=jnp.int32) * _VOCAB
    flat_idx = (x_cat.astype(jnp.int32) + offs[None, :]).reshape(
        _NW, _N_CHUNKS, _CHUNK
    )
    out = _sc_gather(padded, flat_idx)
    return out[:, :_D].reshape(_BATCH, _NUM_FIELDS * _D)


# speedup vs baseline: 5.1971x; 1.0017x over previous
import functools

import jax
import jax.numpy as jnp
from jax import lax
from jax.experimental import pallas as pl
from jax.experimental.pallas import tpu as pltpu
from jax.experimental.pallas import tpu_sc as plsc

_NUM_FIELDS = 26
_VOCAB = 100000
_D = 100
_DP = 128
_BATCH = 16384
_B_TOTAL = _BATCH * _NUM_FIELDS
_NC = 2
_NS = 16
_NW = _NC * _NS
_ROWS_PER_W = _B_TOTAL // _NW
_CHUNK = 128
_N_CHUNKS = _ROWS_PER_W // _CHUNK

_mesh = plsc.VectorSubcoreMesh(core_axis_name="c", subcore_axis_name="s")


@functools.partial(
    pl.kernel,
    out_type=jax.ShapeDtypeStruct((_B_TOTAL, _DP), jnp.float32),
    mesh=_mesh,
    scratch_types=[
        pltpu.VMEM((_N_CHUNKS, _CHUNK), jnp.int32),
        pltpu.VMEM((2, _CHUNK, _DP), jnp.float32),
        pltpu.SemaphoreType.DMA,
        pltpu.SemaphoreType.DMA,
    ],
    compiler_params=pltpu.CompilerParams(skip_device_barrier=True),
)
def _sc_gather(table_hbm, idx_hbm, out_hbm, idx_v, rows_v, g0, g1):
    wid = lax.axis_index("s") * _NC + lax.axis_index("c")
    base = wid * _ROWS_PER_W
    pltpu.sync_copy(idx_hbm.at[wid], idx_v)

    gsems = (g0, g1)

    def _gather(j, t):
        pltpu.async_copy(table_hbm.at[idx_v.at[j]], rows_v.at[t], gsems[t])

    def _wait(t):
        pltpu.make_async_copy(
            table_hbm.at[pl.ds(0, _CHUNK)], rows_v.at[t], gsems[t]
        ).wait()

    def _store(j, t):
        pltpu.sync_copy(rows_v.at[t], out_hbm.at[pl.ds(base + j * _CHUNK, _CHUNK)])

    _gather(0, 0)

    @pl.loop(0, _N_CHUNKS, step=2)
    def _chunk(j):
        _gather(j + 1, 1)
        _wait(0)
        _store(j, 0)

        @pl.when(j + 2 < _N_CHUNKS)
        def _():
            _gather(j + 2, 0)

        _wait(1)
        _store(j + 1, 1)


_VBLK = 32768
_NVB = (_VOCAB + _VBLK - 1) // _VBLK


def _tr_body(src_ref, dst_ref):
    dst_ref[0, :, : _D] = jnp.swapaxes(src_ref[0], 0, 1)


_tc_transpose = pl.pallas_call(
    _tr_body,
    out_shape=jax.ShapeDtypeStruct((_NUM_FIELDS, _VOCAB, _DP), jnp.float32),
    grid=(_NUM_FIELDS, _NVB),
    in_specs=[pl.BlockSpec((1, _D, _VBLK), lambda f, j: (f, 0, j))],
    out_specs=pl.BlockSpec((1, _VBLK, _DP), lambda f, j: (f, j, 0)),
)


def kernel(x_cat, tables):
    tables_t = jnp.transpose(tables, (0, 2, 1))
    padded = _tc_transpose(tables_t).reshape(_NUM_FIELDS * _VOCAB, _DP)
    offs = jnp.arange(_NUM_FIELDS, dtype=jnp.int32) * _VOCAB
    flat_idx = (x_cat.astype(jnp.int32) + offs[None, :]).reshape(
        _NW, _N_CHUNKS, _CHUNK
    )
    out = _sc_gather(padded, flat_idx)
    return out[:, :_D].reshape(_BATCH, _NUM_FIELDS * _D)
